# baseline (speedup 1.0000x reference)
import functools

import jax
import jax.numpy as jnp
from jax import lax
from jax.experimental import pallas as pl
from jax.experimental.pallas import tpu as pltpu
from jax.experimental.pallas import tpu_sc as plsc

D = 128
D_EDGE = 4
NC = 2
NS = 16
L = 16
NW = NC * NS
CHUNK = 64
AGG_PAD_ROWS = 10112


def _pack_bf16_halves(p):
    pr = p.astype(jnp.bfloat16).astype(jnp.float32)
    u = jax.lax.bitcast_convert_type(pr, jnp.uint32)
    lo = u[:, : D // 2] >> 16
    hi = u[:, D // 2:] & jnp.uint32(0xFFFF0000)
    return jax.lax.bitcast_convert_type(lo | hi, jnp.int32)


def _proj_body(x_ref, w_ref, b1_ref, pt_ref):
    n = x_ref.shape[0]
    xv = x_ref[...]
    p = jnp.dot(xv, w_ref[...], preferred_element_type=jnp.float32)
    pt = jnp.concatenate(
        [_pack_bf16_halves(p[:, :D]),
         _pack_bf16_halves(p[:, D:] + b1_ref[...])], axis=1)
    pt_ref[pl.ds(0, n), :] = jax.lax.bitcast_convert_type(pt, jnp.float32)


def _eproj_body(ea_ref, w_ref, o_ref):
    o_ref[...] = jnp.dot(ea_ref[...], w_ref[...],
                         preferred_element_type=jnp.float32)


def _final_body(x_ref, a0_ref, a1_ref, w2_ref, b2_ref, o_ref):
    xv = x_ref[...]
    a = a0_ref[...] + a1_ref[...]
    h = jnp.dot(xv, w2_ref[:D, :], preferred_element_type=jnp.float32)
    h = h + jnp.dot(a, w2_ref[D:, :], preferred_element_type=jnp.float32)
    h = h + b2_ref[...]
    o_ref[...] = jnp.maximum(h, 0.0) + xv


def _make_sc_edge(cpt: int, n_tab: int):
    mesh = plsc.VectorSubcoreMesh(core_axis_name="c", subcore_axis_name="s")

    @functools.partial(
        pl.kernel,
        mesh=mesh,
        compiler_params=pltpu.CompilerParams(needs_layout_passes=False),
        out_type=jax.ShapeDtypeStruct((NC, AGG_PAD_ROWS, D), jnp.float32),
        scratch_types=[
            pltpu.VMEM((2, CHUNK), jnp.int32),
            pltpu.VMEM((2, CHUNK), jnp.int32),
            pltpu.VMEM((2, CHUNK), jnp.int32),
            pltpu.VMEM((2, CHUNK, D), jnp.float32),
            pltpu.VMEM((2, CHUNK, D), jnp.float32),
            pltpu.VMEM((2, CHUNK, D), jnp.float32),
            pltpu.VMEM_SHARED((AGG_PAD_ROWS, D), jnp.float32),
            pltpu.SemaphoreType.DMA,
            pltpu.SemaphoreType.DMA,
            pltpu.SemaphoreType.DMA,
            pltpu.SemaphoreType.DMA,
            pltpu.SemaphoreType.DMA,
            pltpu.SemaphoreType.DMA,
        ],
    )
    def sc_edge(src_hbm, dst_hbm, pt_hbm, eproj_hbm, zeros_hbm,
                out_hbm, sidx, didx, didx_s, abuf, bbuf, ebuf, agg_sh,
                gsem0, gsem1, isem0, isem1, ssem0, ssem1):
        c = lax.axis_index("c")
        s = lax.axis_index("s")
        wid = c * NS + s
        gsems = (gsem0, gsem1)
        isems = (isem0, isem1)
        ssems = (ssem0, ssem1)
        zr = AGG_PAD_ROWS // NS
        pltpu.sync_copy(zeros_hbm.at[pl.ds(s * zr, zr)],
                        agg_sh.at[pl.ds(s * zr, zr)])

        def idx_issue(ch, b):
            base = (wid * cpt + ch) * CHUNK
            pltpu.async_copy(src_hbm.at[pl.ds(base, CHUNK)], sidx.at[b],
                             isems[b])
            pltpu.async_copy(dst_hbm.at[pl.ds(base, CHUNK)], didx.at[b],
                             isems[b])

        def idx_wait(b):
            pltpu.make_async_copy(src_hbm.at[pl.ds(0, CHUNK)], sidx.at[b],
                                  isems[b]).wait()
            pltpu.make_async_copy(dst_hbm.at[pl.ds(0, CHUNK)], didx.at[b],
                                  isems[b]).wait()

        def gather_issue(ch, b):
            ebase = (wid * cpt + ch) * CHUNK
            pltpu.async_copy(pt_hbm.at[sidx.at[b]], abuf.at[b], gsems[b])
            pltpu.async_copy(pt_hbm.at[didx.at[b]], bbuf.at[b], gsems[b])
            pltpu.async_copy(eproj_hbm.at[pl.ds(ebase, CHUNK)],
                             ebuf.at[b], gsems[b])

        def gather_wait(b):
            pltpu.make_async_copy(pt_hbm.at[sidx.at[b]], abuf.at[b],
                                  gsems[b]).wait()
            pltpu.make_async_copy(pt_hbm.at[didx.at[b]], bbuf.at[b],
                                  gsems[b]).wait()
            pltpu.make_async_copy(eproj_hbm.at[pl.ds(0, CHUNK)],
                                  ebuf.at[b], gsems[b]).wait()

        def scatter_wait(b):
            pltpu.make_async_copy(abuf.at[b], agg_sh.at[didx_s.at[b]],
                                  ssems[b]).wait()

        base0 = wid * cpt * CHUNK
        pltpu.sync_copy(src_hbm.at[pl.ds(base0, CHUNK)], sidx.at[0])
        pltpu.sync_copy(dst_hbm.at[pl.ds(base0, CHUNK)], didx.at[0])
        idx_issue(1, 1)
        gather_issue(0, 0)

        def body(g, carry):
            for b in range(2):
                ch = 2 * g + b
                gather_wait(b)

                @pl.when(jnp.logical_and(ch >= 1, ch + 1 < cpt))
                def _():
                    scatter_wait(1 - b)

                @pl.when(ch + 1 < cpt)
                def _():
                    idx_wait(1 - b)
                    gather_issue(ch + 1, 1 - b)

                zero = jnp.zeros((L,), jnp.float32)

                def row_body(r, carry2):
                    for k in range(D // (2 * L)):
                        a = plsc.bitcast(
                            abuf[b, r, pl.ds(k * L, L)], jnp.bfloat16)
                        bb = plsc.bitcast(
                            bbuf[b, r, pl.ds(D // 2 + k * L, L)],
                            jnp.bfloat16)
                        s = a + bb
                        si = plsc.bitcast(s, jnp.int32)
                        lo = plsc.bitcast(si << 16, jnp.float32)
                        hi = plsc.bitcast(si & jnp.int32(-65536),
                                          jnp.float32)
                        lo = lo + ebuf[b, r, pl.ds(k * L, L)]
                        hi = hi + ebuf[b, r, pl.ds(D // 2 + k * L, L)]
                        abuf[b, r, pl.ds(k * L, L)] = jnp.maximum(lo, zero)
                        abuf[b, r, pl.ds(D // 2 + k * L, L)] = (
                            jnp.maximum(hi, zero))
                    return carry2

                lax.fori_loop(0, CHUNK, row_body, 0, unroll=2)
                for k in range(CHUNK // L):
                    sl = pl.ds(k * L, L)
                    didx_s[b, sl] = didx[b, sl]
                pltpu.async_copy(abuf.at[b], agg_sh.at[didx_s.at[b]],
                                 ssems[b], add=True)

                @pl.when(ch + 2 < cpt)
                def _():
                    idx_issue(ch + 2, b)

            return carry

        lax.fori_loop(0, cpt // 2, body, 0)
        scatter_wait(0)
        scatter_wait(1)
        plsc.subcore_barrier()
        pltpu.sync_copy(agg_sh.at[pl.ds(s * zr, zr)],
                        out_hbm.at[c, pl.ds(s * zr, zr)])

    return sc_edge


def kernel(x, edge_index, edge_attr, W1, b1, W2, b2):
    n_nodes = x.shape[0]
    n_edges = edge_index.shape[1]

    cpt = -(-n_edges // (NW * CHUNK))
    cpt = cpt + (cpt % 2)
    e_pad = NW * cpt * CHUNK
    pad = e_pad - n_edges
    src = edge_index[0].astype(jnp.int32)
    dst = edge_index[1].astype(jnp.int32)
    src_p = jnp.concatenate([src, jnp.full((pad,), n_nodes, jnp.int32)])
    dst_p = jnp.concatenate([dst, jnp.full((pad,), n_nodes, jnp.int32)])
    zeros = jnp.zeros((AGG_PAD_ROWS, D), jnp.float32)

    n_tab = n_nodes + 8
    pt = pl.pallas_call(
        _proj_body,
        out_shape=jax.ShapeDtypeStruct((n_tab, D), jnp.float32),
    )(x, jnp.concatenate([W1[:D, :], W1[D:2 * D, :]], axis=1),
      b1.reshape(1, D))

    eblk = 2048
    n_eblk = e_pad // eblk
    last_in = (n_edges - 1) // eblk
    eproj = pl.pallas_call(
        _eproj_body,
        grid=(n_eblk,),
        in_specs=[
            pl.BlockSpec((eblk, D_EDGE),
                         lambda i: (jnp.minimum(i, last_in), 0)),
            pl.BlockSpec((D_EDGE, D), lambda i: (0, 0)),
        ],
        out_specs=pl.BlockSpec((eblk, D), lambda i: (i, 0)),
        out_shape=jax.ShapeDtypeStruct((e_pad, D), jnp.float32),
    )(edge_attr, W1[2 * D:, :])

    agg_parts = _make_sc_edge(cpt, n_tab)(src_p, dst_p, pt, eproj, zeros)

    nblk = 1000
    out = pl.pallas_call(
        _final_body,
        grid=(n_nodes // nblk,),
        in_specs=[
            pl.BlockSpec((nblk, D), lambda i: (i, 0)),
            pl.BlockSpec((nblk, D), lambda i: (i, 0)),
            pl.BlockSpec((nblk, D), lambda i: (i, 0)),
            pl.BlockSpec((2 * D, D), lambda i: (0, 0)),
            pl.BlockSpec((1, D), lambda i: (0, 0)),
        ],
        out_specs=pl.BlockSpec((nblk, D), lambda i: (i, 0)),
        out_shape=jax.ShapeDtypeStruct((n_nodes, D), jnp.float32),
    )(x, agg_parts[0, :n_nodes], agg_parts[1, :n_nodes],
      W2, b2.reshape(1, D))
    return out


# speedup vs baseline: 1.0817x; 1.0817x over previous
import functools

import jax
import jax.numpy as jnp
from jax import lax
from jax.experimental import pallas as pl
from jax.experimental.pallas import tpu as pltpu
from jax.experimental.pallas import tpu_sc as plsc

D = 128
D_EDGE = 4
NC = 2
NS = 16
L = 16
NW = NC * NS
CHUNK = 64
AGG_PAD_ROWS = 10112


def _pack_bf16_halves(p):
    pr = p.astype(jnp.bfloat16).astype(jnp.float32)
    u = jax.lax.bitcast_convert_type(pr, jnp.uint32)
    lo = u[:, : D // 2] >> 16
    hi = u[:, D // 2:] & jnp.uint32(0xFFFF0000)
    return jax.lax.bitcast_convert_type(lo | hi, jnp.int32)


def _proj_body(x_ref, w_ref, b1_ref, pt_ref):
    n = x_ref.shape[0]
    xv = x_ref[...]
    p = jnp.dot(xv, w_ref[...], preferred_element_type=jnp.float32)
    pt = jnp.concatenate(
        [_pack_bf16_halves(p[:, :D]),
         _pack_bf16_halves(p[:, D:] + b1_ref[...])], axis=1)
    pt_ref[pl.ds(0, n), :] = jax.lax.bitcast_convert_type(pt, jnp.float32)


def _eproj_body(ea_ref, w_ref, o_ref):
    o_ref[...] = jnp.dot(ea_ref[...], w_ref[...],
                         preferred_element_type=jnp.float32)


def _final_body(x_ref, a0_ref, a1_ref, w2_ref, b2_ref, o_ref):
    xv = x_ref[...]
    a = a0_ref[...] + a1_ref[...]
    h = jnp.dot(xv, w2_ref[:D, :], preferred_element_type=jnp.float32)
    h = h + jnp.dot(a, w2_ref[D:, :], preferred_element_type=jnp.float32)
    h = h + b2_ref[...]
    o_ref[...] = jnp.maximum(h, 0.0) + xv


def _make_sc_edge(cpt: int, n_tab: int):
    mesh = plsc.VectorSubcoreMesh(core_axis_name="c", subcore_axis_name="s")

    @functools.partial(
        pl.kernel,
        mesh=mesh,
        compiler_params=pltpu.CompilerParams(needs_layout_passes=False),
        out_type=jax.ShapeDtypeStruct((NC, AGG_PAD_ROWS, D), jnp.float32),
        scratch_types=[
            pltpu.VMEM((2, 2 * CHUNK), jnp.int32),
            pltpu.VMEM((2, CHUNK), jnp.int32),
            pltpu.VMEM((2, CHUNK, D), jnp.float32),
            pltpu.VMEM((2, CHUNK, D), jnp.float32),
            pltpu.VMEM((2, CHUNK, D), jnp.float32),
            pltpu.VMEM_SHARED((AGG_PAD_ROWS, D), jnp.float32),
            pltpu.SemaphoreType.DMA,
            pltpu.SemaphoreType.DMA,
            pltpu.SemaphoreType.DMA,
            pltpu.SemaphoreType.DMA,
            pltpu.SemaphoreType.DMA,
            pltpu.SemaphoreType.DMA,
        ],
    )
    def sc_edge(sd_hbm, pt_hbm, eproj_hbm, zeros_hbm,
                out_hbm, sdbuf, didx_s, abuf, bbuf, ebuf, agg_sh,
                gsem0, gsem1, isem0, isem1, ssem0, ssem1):
        c = lax.axis_index("c")
        s = lax.axis_index("s")
        wid = c * NS + s
        gsems = (gsem0, gsem1)
        isems = (isem0, isem1)
        ssems = (ssem0, ssem1)
        zr = AGG_PAD_ROWS // NS
        pltpu.sync_copy(zeros_hbm.at[pl.ds(s * zr, zr)],
                        agg_sh.at[pl.ds(s * zr, zr)])

        def idx_issue(ch, b):
            pltpu.async_copy(sd_hbm.at[wid * cpt + ch], sdbuf.at[b],
                             isems[b])

        def idx_wait(b):
            pltpu.make_async_copy(sd_hbm.at[0], sdbuf.at[b],
                                  isems[b]).wait()

        def gather_issue(ch, b):
            ebase = (wid * cpt + ch) * CHUNK
            pltpu.async_copy(pt_hbm.at[sdbuf.at[b, pl.ds(0, CHUNK)]],
                             abuf.at[b], gsems[b])
            pltpu.async_copy(pt_hbm.at[sdbuf.at[b, pl.ds(CHUNK, CHUNK)]],
                             bbuf.at[b], gsems[b])
            pltpu.async_copy(eproj_hbm.at[pl.ds(ebase, CHUNK)],
                             ebuf.at[b], gsems[b])

        def gather_wait(b):
            pltpu.make_async_copy(pt_hbm.at[sdbuf.at[b, pl.ds(0, CHUNK)]],
                                  abuf.at[b], gsems[b]).wait()
            pltpu.make_async_copy(pt_hbm.at[sdbuf.at[b, pl.ds(CHUNK, CHUNK)]],
                                  bbuf.at[b], gsems[b]).wait()
            pltpu.make_async_copy(eproj_hbm.at[pl.ds(0, CHUNK)],
                                  ebuf.at[b], gsems[b]).wait()

        def scatter_wait(b):
            pltpu.make_async_copy(abuf.at[b], agg_sh.at[didx_s.at[b]],
                                  ssems[b]).wait()

        pltpu.sync_copy(sd_hbm.at[wid * cpt], sdbuf.at[0])
        idx_issue(1, 1)
        gather_issue(0, 0)

        def body(g, carry):
            for b in range(2):
                ch = 2 * g + b
                gather_wait(b)

                @pl.when(jnp.logical_and(ch >= 1, ch + 1 < cpt))
                def _():
                    scatter_wait(1 - b)

                @pl.when(ch + 1 < cpt)
                def _():
                    idx_wait(1 - b)
                    gather_issue(ch + 1, 1 - b)

                zero = jnp.zeros((L,), jnp.float32)

                def row_body(r, carry2):
                    for k in range(D // (2 * L)):
                        a = plsc.bitcast(
                            abuf[b, r, pl.ds(k * L, L)], jnp.bfloat16)
                        bb = plsc.bitcast(
                            bbuf[b, r, pl.ds(D // 2 + k * L, L)],
                            jnp.bfloat16)
                        s = a + bb
                        si = plsc.bitcast(s, jnp.int32)
                        lo = plsc.bitcast(si << 16, jnp.float32)
                        hi = plsc.bitcast(si & jnp.int32(-65536),
                                          jnp.float32)
                        lo = lo + ebuf[b, r, pl.ds(k * L, L)]
                        hi = hi + ebuf[b, r, pl.ds(D // 2 + k * L, L)]
                        abuf[b, r, pl.ds(k * L, L)] = jnp.maximum(lo, zero)
                        abuf[b, r, pl.ds(D // 2 + k * L, L)] = (
                            jnp.maximum(hi, zero))
                    return carry2

                lax.fori_loop(0, CHUNK, row_body, 0, unroll=2)
                for k in range(CHUNK // L):
                    didx_s[b, pl.ds(k * L, L)] = sdbuf[b, pl.ds(CHUNK + k * L, L)]
                pltpu.async_copy(abuf.at[b], agg_sh.at[didx_s.at[b]],
                                 ssems[b], add=True)

                @pl.when(ch + 2 < cpt)
                def _():
                    idx_issue(ch + 2, b)

            return carry

        lax.fori_loop(0, cpt // 2, body, 0)
        scatter_wait(0)
        scatter_wait(1)
        plsc.subcore_barrier()
        pltpu.sync_copy(agg_sh.at[pl.ds(s * zr, zr)],
                        out_hbm.at[c, pl.ds(s * zr, zr)])

    return sc_edge


def kernel(x, edge_index, edge_attr, W1, b1, W2, b2):
    n_nodes = x.shape[0]
    n_edges = edge_index.shape[1]

    cpt = -(-n_edges // (NW * CHUNK))
    cpt = cpt + (cpt % 2)
    e_pad = NW * cpt * CHUNK
    pad = e_pad - n_edges
    src = edge_index[0].astype(jnp.int32)
    dst = edge_index[1].astype(jnp.int32)
    src_p = jnp.concatenate([src, jnp.full((pad,), n_nodes, jnp.int32)])
    dst_p = jnp.concatenate([dst, jnp.full((pad,), n_nodes, jnp.int32)])
    sd = jnp.concatenate([src_p.reshape(-1, CHUNK),
                          dst_p.reshape(-1, CHUNK)], axis=1)
    zeros = jnp.zeros((AGG_PAD_ROWS, D), jnp.float32)

    n_tab = n_nodes + 8
    pt = pl.pallas_call(
        _proj_body,
        out_shape=jax.ShapeDtypeStruct((n_tab, D), jnp.float32),
    )(x, jnp.concatenate([W1[:D, :], W1[D:2 * D, :]], axis=1),
      b1.reshape(1, D))

    eblk = 4096
    n_eblk = e_pad // eblk
    last_in = (n_edges - 1) // eblk
    eproj = pl.pallas_call(
        _eproj_body,
        grid=(n_eblk,),
        in_specs=[
            pl.BlockSpec((eblk, D_EDGE),
                         lambda i: (jnp.minimum(i, last_in), 0)),
            pl.BlockSpec((D_EDGE, D), lambda i: (0, 0)),
        ],
        out_specs=pl.BlockSpec((eblk, D), lambda i: (i, 0)),
        out_shape=jax.ShapeDtypeStruct((e_pad, D), jnp.float32),
    )(edge_attr, W1[2 * D:, :])

    agg_parts = _make_sc_edge(cpt, n_tab)(sd, pt, eproj, zeros)

    nblk = 1000
    out = pl.pallas_call(
        _final_body,
        grid=(n_nodes // nblk,),
        in_specs=[
            pl.BlockSpec((nblk, D), lambda i: (i, 0)),
            pl.BlockSpec((nblk, D), lambda i: (i, 0)),
            pl.BlockSpec((nblk, D), lambda i: (i, 0)),
            pl.BlockSpec((2 * D, D), lambda i: (0, 0)),
            pl.BlockSpec((1, D), lambda i: (0, 0)),
        ],
        out_specs=pl.BlockSpec((nblk, D), lambda i: (i, 0)),
        out_shape=jax.ShapeDtypeStruct((n_nodes, D), jnp.float32),
    )(x, agg_parts[0, :n_nodes], agg_parts[1, :n_nodes],
      W2, b2.reshape(1, D))
    return out
